# baseline (speedup 1.0000x reference)
import math

import jax
import jax.numpy as jnp
from jax.experimental import pallas as pl


def _body(off_ref, blk_ref, q_ref, k_ref, v_ref, out_ref, attn_ref, *, L, D, U):
    K = k_ref[0, 0]
    V = v_ref[0, 0]

    LC = 512
    m_parts = []
    for c in range(L // LC):
        Qc = q_ref[0, 0, c * LC:(c + 1) * LC, :]
        offs = off_ref[c * LC:(c + 1) * LC, :]
        blks = blk_ref[c * LC:(c + 1) * LC, :]
        QK = jnp.zeros((LC, U), jnp.float32)
        for blk in range(L // 128):
            Sb = jnp.dot(Qc, K[blk * 128:(blk + 1) * 128].T)
            gb = jnp.take_along_axis(Sb, offs, axis=1)
            QK = jnp.where(blks == blk, gb, QK)
        mc = jnp.max(QK, axis=1) - jnp.sum(QK, axis=1) * (1.0 / L)
        m_parts.append(mc.reshape(1, LC))
    M = jnp.concatenate(m_parts, axis=1)

    iota = jax.lax.broadcasted_iota(jnp.int32, (1, L), 1)
    sel_rows = []
    mcur = M
    for _ in range(U):
        mx = jnp.max(mcur)
        hit = mcur == mx
        idx = jnp.min(jnp.where(hit, iota, L))
        row = (iota == idx)
        sel_rows.append(row.astype(jnp.float32))
        mcur = jnp.where(row, -jnp.inf, mcur)

    oh = jnp.concatenate(sel_rows, axis=0)

    Qr = jax.lax.dot(oh, q_ref[0, 0], precision=jax.lax.Precision.HIGHEST)
    scores = jnp.dot(Qr, K.T) * (1.0 / math.sqrt(D))
    smax = jnp.max(scores, axis=1, keepdims=True)
    e = jnp.exp(scores - smax)
    a = e / jnp.sum(e, axis=1, keepdims=True)
    attn_ref[0, 0] = a

    upd = jnp.dot(a, V)

    repl = jax.lax.dot(oh.T, upd, precision=jax.lax.Precision.HIGHEST)
    covered = jnp.sum(oh, axis=0).reshape(L, 1) > 0.0
    meanV = jnp.mean(V, axis=0, keepdims=True)
    out_ref[0, 0] = jnp.where(covered, repl, jnp.broadcast_to(meanV, (L, D)))


def kernel(queries, keys, values):
    B, L, H, D = queries.shape
    U = int(math.ceil(math.log(L)))

    Qh = jnp.transpose(queries, (0, 2, 1, 3))
    Kh = jnp.transpose(keys, (0, 2, 1, 3))
    Vh = jnp.transpose(values, (0, 2, 1, 3))

    idx = jax.random.randint(jax.random.key(42), (L, U), 0, L)
    off = (idx % 128).astype(jnp.int32)
    blk = (idx // 128).astype(jnp.int32)

    import functools
    body = functools.partial(_body, L=L, D=D, U=U)

    head_spec = pl.BlockSpec((1, 1, L, D), lambda i: (i // H, i % H, 0, 0))
    out_t, attn = pl.pallas_call(
        body,
        grid=(B * H,),
        in_specs=[
            pl.BlockSpec((L, U), lambda i: (0, 0)),
            pl.BlockSpec((L, U), lambda i: (0, 0)),
            head_spec,
            head_spec,
            head_spec,
        ],
        out_specs=[
            pl.BlockSpec((1, 1, L, D), lambda i: (i // H, i % H, 0, 0)),
            pl.BlockSpec((1, 1, U, L), lambda i: (i // H, i % H, 0, 0)),
        ],
        out_shape=[
            jax.ShapeDtypeStruct((B, H, L, D), jnp.float32),
            jax.ShapeDtypeStruct((B, H, U, L), jnp.float32),
        ],
    )(off, blk, Qh, Kh, Vh)

    out = jnp.transpose(out_t, (0, 2, 1, 3))
    return (out, attn)


# speedup vs baseline: 1.0015x; 1.0015x over previous
import math

import jax
import jax.numpy as jnp
from jax.experimental import pallas as pl


def _body(off_ref, blk_ref, q_ref, k_ref, v_ref, out_ref, attn_ref, *, L, D, U):
    K = k_ref[0, 0]
    V = v_ref[0, 0]

    LC = 512
    m_parts = []
    for c in range(L // LC):
        Qc = q_ref[0, 0, c * LC:(c + 1) * LC, :]
        offs = off_ref[c * LC:(c + 1) * LC, :]
        blks = blk_ref[c * LC:(c + 1) * LC, :]
        S = jnp.dot(Qc, K.T)
        QK = jnp.zeros((LC, U), jnp.float32)
        for blk in range(L // 128):
            gb = jnp.take_along_axis(
                S[:, blk * 128:(blk + 1) * 128], offs, axis=1)
            QK = jnp.where(blks == blk, gb, QK)
        mc = jnp.max(QK, axis=1) - jnp.sum(QK, axis=1) * (1.0 / L)
        m_parts.append(mc.reshape(1, LC))
    M = jnp.concatenate(m_parts, axis=1)

    iota = jax.lax.broadcasted_iota(jnp.int32, (1, L), 1)
    sel_rows = []
    mcur = M
    for _ in range(U):
        mx = jnp.max(mcur)
        hit = mcur == mx
        idx = jnp.min(jnp.where(hit, iota, L))
        row = (iota == idx)
        sel_rows.append(row.astype(jnp.float32))
        mcur = jnp.where(row, -jnp.inf, mcur)

    oh = jnp.concatenate(sel_rows, axis=0)

    Qr = jax.lax.dot(oh, q_ref[0, 0], precision=jax.lax.Precision.HIGHEST)
    scores = jnp.dot(Qr, K.T) * (1.0 / math.sqrt(D))
    smax = jnp.max(scores, axis=1, keepdims=True)
    e = jnp.exp(scores - smax)
    a = e / jnp.sum(e, axis=1, keepdims=True)
    attn_ref[0, 0] = a

    upd = jnp.dot(a, V)

    repl = jax.lax.dot(oh.T, upd, precision=jax.lax.Precision.HIGHEST)
    covered = jnp.sum(oh, axis=0).reshape(L, 1) > 0.0
    meanV = jnp.mean(V, axis=0, keepdims=True)
    out_ref[0, 0] = jnp.where(covered, repl, jnp.broadcast_to(meanV, (L, D)))


def kernel(queries, keys, values):
    B, L, H, D = queries.shape
    U = int(math.ceil(math.log(L)))

    Qh = jnp.transpose(queries, (0, 2, 1, 3))
    Kh = jnp.transpose(keys, (0, 2, 1, 3))
    Vh = jnp.transpose(values, (0, 2, 1, 3))

    idx = jax.random.randint(jax.random.key(42), (L, U), 0, L)
    off = (idx % 128).astype(jnp.int32)
    blk = (idx // 128).astype(jnp.int32)

    import functools
    body = functools.partial(_body, L=L, D=D, U=U)

    head_spec = pl.BlockSpec((1, 1, L, D), lambda i: (i // H, i % H, 0, 0))
    out_t, attn = pl.pallas_call(
        body,
        grid=(B * H,),
        in_specs=[
            pl.BlockSpec((L, U), lambda i: (0, 0)),
            pl.BlockSpec((L, U), lambda i: (0, 0)),
            head_spec,
            head_spec,
            head_spec,
        ],
        out_specs=[
            pl.BlockSpec((1, 1, L, D), lambda i: (i // H, i % H, 0, 0)),
            pl.BlockSpec((1, 1, U, L), lambda i: (i // H, i % H, 0, 0)),
        ],
        out_shape=[
            jax.ShapeDtypeStruct((B, H, L, D), jnp.float32),
            jax.ShapeDtypeStruct((B, H, U, L), jnp.float32),
        ],
    )(off, blk, Qh, Kh, Vh)

    out = jnp.transpose(out_t, (0, 2, 1, 3))
    return (out, attn)


# speedup vs baseline: 1.1886x; 1.1868x over previous
import functools
import math

import jax
import jax.numpy as jnp
from jax import lax
from jax.experimental import pallas as pl
from jax.experimental.pallas import tpu as pltpu
from jax.experimental.pallas import tpu_sc as plsc

_QCH = 64


def _sc_m_body(idxc_hbm, qflat_hbm, kflat_hbm, m_hbm,
               sidx_v, qidx_v, qrows_v, krows_v, m_v, sem, *, B, L, H, D, U):
    NC = 2
    wid = lax.axis_index("s") * NC + lax.axis_index("c")
    b = wid // H
    h = wid % H
    base = b * (L * H) + h
    iota16 = lax.iota(jnp.int32, 16)
    iotaH = iota16 * H
    nv = D // 16
    rolls = [jnp.bitwise_and(iota16 + sh, 15).reshape(16, 1) for sh in (8, 4, 2, 1)]
    _dnums = lax.GatherDimensionNumbers(
        offset_dims=(), collapsed_slice_dims=(0,), start_index_map=(0,))

    def _lanesum(t):
        for r in rolls:
            t = t + lax.gather(t, r, _dnums, (1,),
                               mode=lax.GatherScatterMode.PROMISE_IN_BOUNDS)
        return t

    def _bf16r(v):
        yi = lax.bitcast_convert_type(v, jnp.int32)
        rnd = lax.shift_right_logical(yi, 16) & 1
        yi = (yi + (32767 + rnd)) & (-65536)
        return lax.bitcast_convert_type(yi, jnp.float32)

    def chunk_body(c, carry):
        lo = c * _QCH
        pltpu.sync_copy(idxc_hbm.at[c], sidx_v)
        for s in range(U):
            for g in range(_QCH // 16):
                sl = pl.ds(g * 16, 16)
                sidx_v[s, sl] = sidx_v[s, sl] + base
        qoff = base + lo * H
        for g in range(_QCH // 16):
            qidx_v[pl.ds(g * 16, 16)] = qoff + g * 16 * H + iotaH
        cps = [pltpu.async_copy(qflat_hbm.at[qidx_v], qrows_v, sem)]
        for s in range(U):
            cps.append(
                pltpu.async_copy(kflat_hbm.at[sidx_v.at[s]], krows_v.at[s], sem))
        for cp in cps:
            cp.wait()

        def rnd_body(qi, carry2):
            for t in range(nv):
                sl = pl.ds(t * 16, 16)
                qrows_v[qi, sl] = _bf16r(qrows_v[qi, sl])
                for s in range(U):
                    krows_v[s, qi, sl] = _bf16r(krows_v[s, qi, sl])
            return carry2

        lax.fori_loop(0, _QCH, rnd_body, 0)

        def sub(g, carry2):
            macc = jnp.zeros((16,), jnp.float32)
            for j in range(16):
                qi = g * 16 + j
                qv = [qrows_v[qi, pl.ds(t * 16, 16)] for t in range(nv)]
                rs = []
                for s in range(U):
                    t0 = qv[0] * krows_v[s, qi, pl.ds(0, 16)]
                    for t in range(1, nv):
                        t0 = t0 + qv[t] * krows_v[s, qi, pl.ds(t * 16, 16)]
                    rs.append(_lanesum(t0))
                rmax = rs[0]
                rsum = rs[0]
                for s in range(1, U):
                    rmax = jnp.maximum(rmax, rs[s])
                    rsum = rsum + rs[s]
                mval = rmax - rsum * (1.0 / L)
                macc = jnp.where(iota16 == j, mval, macc)
            m_v[pl.ds(lo + g * 16, 16)] = macc
            return carry2

        lax.fori_loop(0, _QCH // 16, sub, 0)
        return carry

    lax.fori_loop(0, L // _QCH, chunk_body, 0)
    pltpu.sync_copy(m_v, m_hbm.at[wid])


def _tc_body(mt_ref, q_ref, k_ref, v_ref, out_ref, attn_ref, *, L, D, U):
    K = k_ref[0, 0]
    V = v_ref[0, 0]
    mt = mt_ref[0, 0]

    iota = lax.broadcasted_iota(jnp.int32, (1, L), 1)
    Q = q_ref[0, 0]
    cov = jnp.zeros((1, L), jnp.float32)
    repl = jnp.zeros((L, D), jnp.float32)
    scale = 1.0 / math.sqrt(D)
    for j in range(U):
        rowb = (iota == mt[j])
        row = rowb.astype(jnp.float32)
        cov = cov + row
        qrow = lax.dot(row, Q, precision=lax.Precision.HIGHEST)
        s = jnp.dot(qrow, K.T) * scale
        e = jnp.exp(s - jnp.max(s))
        a = e / jnp.sum(e)
        attn_ref[0, 0, j:j + 1, :] = a
        upd = jnp.dot(a, V)
        repl = repl + lax.dot(row.T, upd, precision=lax.Precision.HIGHEST)

    meanV = jnp.mean(V, axis=0, keepdims=True)
    out_ref[0, 0] = jnp.where(cov.reshape(L, 1) > 0.0, repl,
                              jnp.broadcast_to(meanV, (L, D)))


def kernel(queries, keys, values):
    B, L, H, D = queries.shape
    U = int(math.ceil(math.log(L)))

    idx = jax.random.randint(jax.random.key(42), (L, U), 0, L)
    idxH = (idx * H).astype(jnp.int32)
    idxc = jnp.transpose(idxH.reshape(L // _QCH, _QCH, U), (0, 2, 1))

    qflat = queries.astype(jnp.bfloat16).astype(jnp.float32).reshape(B * L * H, D)
    kflat = keys.astype(jnp.bfloat16).astype(jnp.float32).reshape(B * L * H, D)

    mesh = plsc.VectorSubcoreMesh(core_axis_name="c", subcore_axis_name="s")
    sc_m = functools.partial(
        pl.kernel,
        mesh=mesh,
        compiler_params=pltpu.CompilerParams(use_tc_tiling_on_sc=False),
        out_type=jax.ShapeDtypeStruct((B * H, L), jnp.float32),
        scratch_types=[
            pltpu.VMEM((U, _QCH), jnp.int32),
            pltpu.VMEM((_QCH,), jnp.int32),
            pltpu.VMEM((_QCH, D), jnp.float32),
            pltpu.VMEM((U, _QCH, D), jnp.float32),
            pltpu.VMEM((L,), jnp.float32),
            pltpu.SemaphoreType.DMA,
        ],
    )(functools.partial(_sc_m_body, B=B, L=L, H=H, D=D, U=U))
    m = sc_m(idxc, qflat, kflat)
    mtop = lax.top_k(m, U)[1].astype(jnp.int32)
    mt3 = mtop.reshape(B * H, 1, U)

    Qh = jnp.transpose(queries, (0, 2, 1, 3))
    Kh = jnp.transpose(keys, (0, 2, 1, 3))
    Vh = jnp.transpose(values, (0, 2, 1, 3))

    body = functools.partial(_tc_body, L=L, D=D, U=U)
    head_spec = pl.BlockSpec((1, 1, L, D), lambda i: (i // H, i % H, 0, 0))
    out_t, attn = pl.pallas_call(
        body,
        grid=(B * H,),
        in_specs=[
            pl.BlockSpec((1, 1, U), lambda i: (i, 0, 0)),
            head_spec,
            head_spec,
            head_spec,
        ],
        out_specs=[
            pl.BlockSpec((1, 1, L, D), lambda i: (i // H, i % H, 0, 0)),
            pl.BlockSpec((1, 1, U, L), lambda i: (i // H, i % H, 0, 0)),
        ],
        out_shape=[
            jax.ShapeDtypeStruct((B, H, L, D), jnp.float32),
            jax.ShapeDtypeStruct((B, H, U, L), jnp.float32),
        ],
    )(mt3, Qh, Kh, Vh)

    out = jnp.transpose(out_t, (0, 2, 1, 3))
    return (out, attn)
